# baseline (speedup 1.0000x reference)
import functools
import math

import jax
import jax.numpy as jnp
from jax import lax
from jax.experimental import pallas as pl
from jax.experimental.pallas import tpu as pltpu
from jax.experimental.pallas import tpu_sc as plsc

DM = 64
SCALE = math.sqrt(DM)

NC = 2
NS = 16
NW = NC * NS
L = 16
BW = 128


def _emb_lookup(tr, xt):
    sl, nb = xt.shape
    assert nb == NW * BW

    mesh = plsc.VectorSubcoreMesh(core_axis_name="c", subcore_axis_name="s")

    @functools.partial(
        pl.kernel,
        mesh=mesh,
        out_type=jax.ShapeDtypeStruct((sl, DM, nb), jnp.float32),
        scratch_types=[
            pltpu.VMEM((sl, BW), jnp.int32),
            pltpu.VMEM((sl, BW), jnp.int32),
            pltpu.VMEM((2, BW, 128), jnp.float32),
            pltpu.VMEM((2, DM, BW), jnp.float32),
            pltpu.SemaphoreType.DMA,
            pltpu.SemaphoreType.DMA,
            pltpu.SemaphoreType.DMA,
            pltpu.SemaphoreType.DMA,
        ],
        compiler_params=pltpu.CompilerParams(use_tc_tiling_on_sc=True,
                                             needs_layout_passes=False),
    )
    def k(tr_hbm, xt_hbm, out_hbm, idx_v, ih_v, g_v, o_v, gsem0, gsem1,
          wsem0, wsem1):
        gsems = (gsem0, gsem1)
        wsems = (wsem0, wsem1)
        wid = lax.axis_index("s") * NC + lax.axis_index("c")
        b0 = wid * BW

        pltpu.sync_copy(xt_hbm.at[:, pl.ds(b0, BW)], idx_v)

        def halve(s, carry):
            for kk in range(BW // L):
                c = pl.ds(kk * L, L)
                ih_v[s, c] = lax.shift_right_logical(idx_v[s, c], 1)
            return carry

        lax.fori_loop(0, sl, halve, 0)

        def fire(s, bb):
            pltpu.async_copy(tr_hbm.at[ih_v.at[s]], g_v.at[bb], gsems[bb])

        def drain_g(bb):
            pltpu.make_async_copy(tr_hbm.at[pl.ds(0, BW)],
                                  g_v.at[bb], gsems[bb]).wait()

        def drain_w(bb):
            pltpu.make_async_copy(out_hbm.at[0, :, pl.ds(0, BW)],
                                  o_v.at[bb], wsems[bb]).wait()

        fire(0, 0)

        def pair(t, carry):
            go = t * 2
            for b in (0, 1):
                s = go + b
                nb_ = 1 - b

                @pl.when(s + 1 < sl)
                def _():
                    @pl.when(s >= 1)
                    def _():
                        drain_w(nb_)
                    fire(s + 1, nb_)

                drain_g(b)

                def repack(kk, carry2, _b=b):
                    lanes = jax.lax.iota(jnp.int32, L) + kk * L
                    raw = plsc.load_gather(idx_v, [
                        jnp.full((L,), s, jnp.int32), lanes])
                    colbase = lax.shift_left(
                        lax.bitwise_and(raw, jnp.int32(1)), 6)
                    for d in range(DM):
                        v = plsc.load_gather(g_v.at[_b],
                                             [lanes, colbase + d])
                        o_v[_b, d, pl.ds(kk * L, L)] = v * SCALE
                    return carry2

                lax.fori_loop(0, BW // L, repack, 0)
                pltpu.async_copy(
                    o_v.at[b],
                    out_hbm.at[s, :, pl.ds(b0, BW)],
                    wsems[b],
                )
            return carry

        lax.fori_loop(0, sl // 2, pair, 0)
        drain_w(0)
        drain_w(1)

    return k(tr, xt)


def kernel(x, table):
    tr = table.reshape(500000, 128)
    out = _emb_lookup(tr, x.T)
    return jnp.transpose(out, (2, 0, 1))


# speedup vs baseline: 1.6341x; 1.6341x over previous
import functools
import math

import jax
import jax.numpy as jnp
from jax import lax
from jax.experimental import pallas as pl
from jax.experimental.pallas import tpu as pltpu
from jax.experimental.pallas import tpu_sc as plsc

DM = 64
SCALE = math.sqrt(DM)

NC = 2
NS = 16
NW = NC * NS
L = 16

CB = 4


def _emb_lookup(table, x):
    nb, sl = x.shape
    rows_per_w = nb // NW
    n_chunks = rows_per_w // CB
    assert rows_per_w % CB == 0 and n_chunks % 2 == 0
    g0 = 128
    g1 = sl - g0

    mesh = plsc.VectorSubcoreMesh(core_axis_name="c", subcore_axis_name="s")

    @functools.partial(
        pl.kernel,
        mesh=mesh,
        out_type=jax.ShapeDtypeStruct((nb, sl, DM), jnp.float32),
        scratch_types=[
            pltpu.VMEM((2, CB, sl), jnp.int32),
            pltpu.VMEM((2, CB, sl, DM), jnp.float32),
            pltpu.SemaphoreType.DMA,
            pltpu.SemaphoreType.DMA,
            pltpu.SemaphoreType.DMA,
            pltpu.SemaphoreType.DMA,
        ],
        compiler_params=pltpu.CompilerParams(use_tc_tiling_on_sc=False),
    )
    def k(table_hbm, x_hbm, out_hbm, idx_v, rows_v, gsem0, gsem1, wsem0,
          wsem1):
        gsems = (gsem0, gsem1)
        wsems = (wsem0, wsem1)
        wid = lax.axis_index("s") * NC + lax.axis_index("c")
        base = wid * rows_per_w

        def fire(c, bb):
            b0 = base + c * CB
            pltpu.sync_copy(x_hbm.at[pl.ds(b0, CB)], idx_v.at[bb])
            for r in range(CB):
                pltpu.async_copy(
                    table_hbm.at[idx_v.at[bb, r, pl.ds(0, g0)]],
                    rows_v.at[bb, r, pl.ds(0, g0)],
                    gsems[bb],
                )
                pltpu.async_copy(
                    table_hbm.at[idx_v.at[bb, r, pl.ds(g0, g1)]],
                    rows_v.at[bb, r, pl.ds(g0, g1)],
                    gsems[bb],
                )

        def drain(sem, bb):
            pltpu.make_async_copy(out_hbm.at[pl.ds(0, CB)],
                                  rows_v.at[bb], sem).wait()

        fire(0, 0)

        def pair(t, carry):
            go = t * 2
            for b in (0, 1):
                c = go + b
                nb_ = 1 - b

                @pl.when(c + 1 < n_chunks)
                def _():
                    @pl.when(c >= 1)
                    def _():
                        drain(wsems[nb_], nb_)
                    fire(c + 1, nb_)

                drain(gsems[b], b)

                for r in range(CB):

                    def scale_row(i, cr, _b=b, _r=r):
                        for j in range(DM // L):
                            s = pl.ds(j * L, L)
                            rows_v[_b, _r, i, s] = rows_v[_b, _r, i, s] * SCALE
                        return cr

                    lax.fori_loop(0, sl, scale_row, 0, unroll=4)

                pltpu.async_copy(
                    rows_v.at[b],
                    out_hbm.at[pl.ds(base + c * CB, CB)],
                    wsems[b],
                )
            return carry

        lax.fori_loop(0, n_chunks // 2, pair, 0)
        drain(wsems[0], 0)
        drain(wsems[1], 1)

    return k(table, x)


def kernel(x, table):
    return _emb_lookup(table, x)
